# baseline (speedup 1.0000x reference)
import functools

import jax
import jax.numpy as jnp
from jax import lax
from jax.experimental import pallas as pl
from jax.experimental.pallas import tpu as pltpu
from jax.experimental.pallas import tpu_sc as plsc

_SC_PARAMS = pltpu.CompilerParams(needs_layout_passes=False)


def _make_dedup(D0, K):
    info = plsc.get_sparse_core_info()
    NC, NS, L = info.num_cores, info.num_subcores, info.num_lanes
    NW = NC * NS
    CPW = K // NW
    PIECE = 2048
    NPIECE = K // PIECE
    VPP = PIECE // L
    POS_BITS = max(K - 1, 1).bit_length()
    POS_MASK = (1 << POS_BITS) - 1

    mesh = plsc.VectorSubcoreMesh(core_axis_name="c", subcore_axis_name="s")

    @functools.partial(
        pl.kernel,
        mesh=mesh,
        out_type=jax.ShapeDtypeStruct((K,), jnp.int32),
        compiler_params=_SC_PARAMS,
        scratch_types=[
            pltpu.VMEM((PIECE,), jnp.int32),
            pltpu.VMEM((D0,), jnp.int32),
            pltpu.VMEM((L,), jnp.int32),
            pltpu.VMEM((CPW,), jnp.int32),
            pltpu.VMEM((CPW,), jnp.int32),
        ],
    )
    def dedup(idx_hbm, jsrc_hbm, piece_v, table, shift_v, own_v, jbuf):
        cid = lax.axis_index("c")
        sid = lax.axis_index("s")
        wid = sid * NC + cid
        base = wid * CPW
        lanes = lax.iota(jnp.int32, L)
        last_lane = lanes == (L - 1)
        nxt_lane = jnp.minimum(lanes + 1, L - 1)

        def piece_body(p, _):
            pltpu.sync_copy(idx_hbm.at[pl.ds(p * PIECE, PIECE)], piece_v)

            def vreg_body(v, _):
                iv = piece_v[pl.ds(v * L, L)]
                pos = p * PIECE + v * L + lanes
                comb = (iv << POS_BITS) | pos
                ks, _ = plsc.sort_key_val(comb, comb)
                idx_s = ks >> POS_BITS
                pos_s = ks & POS_MASK
                shift_v[...] = idx_s
                nxt = plsc.load_gather(shift_v, [nxt_lane])
                keep = (idx_s != nxt) | last_lane
                plsc.store_scatter(table, [idx_s], pos_s, mask=keep)
                return 0

            lax.fori_loop(0, VPP, vreg_body, 0)
            return 0

        lax.fori_loop(0, NPIECE, piece_body, 0)

        pltpu.sync_copy(idx_hbm.at[pl.ds(base, CPW)], own_v)

        def src_body(c, _):
            iv = own_v[pl.ds(c * L, L)]
            jbuf[pl.ds(c * L, L)] = plsc.load_gather(table, [iv])
            return 0

        lax.fori_loop(0, CPW // L, src_body, 0)
        pltpu.sync_copy(jbuf, jsrc_hbm.at[pl.ds(base, CPW)])

    return dedup


def _make_transpose(K, R):
    info = plsc.get_sparse_core_info()
    NC, NS, L = info.num_cores, info.num_subcores, info.num_lanes
    NW = NC * NS
    CPW = K // NW
    TCH = 128
    NTCH = CPW // TCH
    RH = R // 2

    mesh = plsc.VectorSubcoreMesh(core_axis_name="c", subcore_axis_name="s")

    @functools.partial(
        pl.kernel,
        mesh=mesh,
        out_type=jax.ShapeDtypeStruct((K, R), jnp.float32),
        compiler_params=_SC_PARAMS,
        scratch_types=[
            pltpu.VMEM((RH, TCH), jnp.float32),
            pltpu.VMEM((TCH, R), jnp.float32),
        ],
    )
    def transpose(updt_hbm, upd_rm_hbm, tin, tout):
        cid = lax.axis_index("c")
        sid = lax.axis_index("s")
        wid = sid * NC + cid
        base = wid * CPW
        lanes = lax.iota(jnp.int32, L)

        def ch_body(ch, _):
            k0 = base + ch * TCH
            for rh in range(2):
                pltpu.sync_copy(
                    updt_hbm.at[pl.ds(rh * RH, RH), pl.ds(k0, TCH)], tin)

                def r_body(r, _):
                    rsplat = jnp.full((L,), rh * RH, jnp.int32) + r

                    def q_body(q, _):
                        vals = tin[r, pl.ds(q * L, L)]
                        plsc.store_scatter(tout, [q * L + lanes, rsplat], vals)
                        return 0

                    lax.fori_loop(0, TCH // L, q_body, 0, unroll=True)
                    return 0

                lax.fori_loop(0, RH, r_body, 0, unroll=4)
            pltpu.sync_copy(tout, upd_rm_hbm.at[pl.ds(k0, TCH)])
            return 0

        lax.fori_loop(0, NTCH, ch_body, 0)

    return transpose


def _make_scatter(D0, K, R):
    info = plsc.get_sparse_core_info()
    NC, NS, L = info.num_cores, info.num_subcores, info.num_lanes
    NW = NC * NS
    CPW = K // NW
    CHUNK = 64
    NCH = CPW // CHUNK

    mesh = plsc.VectorSubcoreMesh(core_axis_name="c", subcore_axis_name="s")

    @functools.partial(
        pl.kernel,
        mesh=mesh,
        out_type=(),
        compiler_params=_SC_PARAMS,
        scratch_types=[
            pltpu.VMEM((CPW,), jnp.int32),
            pltpu.VMEM((NCH, CHUNK), jnp.int32),
            pltpu.VMEM((NCH, CHUNK), jnp.int32),
            pltpu.VMEM((CHUNK, R), jnp.float32),
            pltpu.VMEM((CHUNK, R), jnp.float32),
            pltpu.SemaphoreType.DMA,
            pltpu.SemaphoreType.DMA,
            pltpu.SemaphoreType.DMA,
            pltpu.SemaphoreType.DMA,
        ],
    )
    def scatter(idx_hbm, jsrc_hbm, upd_hbm, out_hbm, stage_v, idx_own, jsrc,
                buf0, buf1, gsem0, gsem1, ssem0, ssem1):
        cid = lax.axis_index("c")
        sid = lax.axis_index("s")
        wid = sid * NC + cid
        base = wid * CPW

        pltpu.sync_copy(idx_hbm.at[pl.ds(base, CPW)], stage_v)

        def repack_idx(c, _):
            def repack_row(q, _):
                idx_own[c, pl.ds(q * L, L)] = stage_v[pl.ds(c * CHUNK + q * L, L)]
                return 0

            lax.fori_loop(0, CHUNK // L, repack_row, 0, unroll=True)
            return 0

        lax.fori_loop(0, NCH, repack_idx, 0)
        pltpu.sync_copy(jsrc_hbm.at[pl.ds(base, CPW)], stage_v)

        def repack_j(c, _):
            def repack_row(q, _):
                jsrc[c, pl.ds(q * L, L)] = stage_v[pl.ds(c * CHUNK + q * L, L)]
                return 0

            lax.fori_loop(0, CHUNK // L, repack_row, 0, unroll=True)
            return 0

        lax.fori_loop(0, NCH, repack_j, 0)

        bufs = (buf0, buf1)
        gsems = (gsem0, gsem1)
        ssems = (ssem0, ssem1)

        def pipe_body(cc, _):
            for b in range(2):
                c = cc * 2 + b

                @pl.when(cc > 0)
                def _():
                    pltpu.make_async_copy(
                        bufs[b], out_hbm.at[idx_own.at[0]], ssems[b]).wait()

                pltpu.async_copy(upd_hbm.at[jsrc.at[c]], bufs[b], gsems[b]).wait()
                pltpu.async_copy(bufs[b], out_hbm.at[idx_own.at[c]], ssems[b])
            return 0

        lax.fori_loop(0, NCH // 2, pipe_body, 0)
        for b in range(2):
            pltpu.make_async_copy(
                bufs[b], out_hbm.at[idx_own.at[0]], ssems[b]).wait()

    return scatter


def kernel(input, indices, update):
    D0, D1, D2 = input.shape
    K = indices.shape[0]
    R = D1 * D2
    jsrc = _make_dedup(D0, K)(indices)
    upd_t = update.transpose(1, 2, 0).reshape(R, K)
    upd_rm = _make_transpose(K, R)(upd_t)
    out_ref = jax.new_ref(input.reshape(D0, R))
    _make_scatter(D0, K, R)(indices, jsrc, upd_rm, out_ref)
    return jax.freeze(out_ref).reshape(D0, D1, D2)


# speedup vs baseline: 1.0916x; 1.0916x over previous
import functools

import jax
import jax.numpy as jnp
from jax import lax
from jax.experimental import pallas as pl
from jax.experimental.pallas import tpu as pltpu
from jax.experimental.pallas import tpu_sc as plsc

_SC_PARAMS = pltpu.CompilerParams(needs_layout_passes=False)


def _make_dedup(D0, K):
    info = plsc.get_sparse_core_info()
    NC, NS, L = info.num_cores, info.num_subcores, info.num_lanes
    NW = NC * NS
    CPW = K // NW
    PIECE = 2048
    NPIECE = K // PIECE
    VPP = PIECE // L
    POS_BITS = max(K - 1, 1).bit_length()
    POS_MASK = (1 << POS_BITS) - 1

    mesh = plsc.VectorSubcoreMesh(core_axis_name="c", subcore_axis_name="s")

    @functools.partial(
        pl.kernel,
        mesh=mesh,
        out_type=jax.ShapeDtypeStruct((K,), jnp.int32),
        compiler_params=_SC_PARAMS,
        scratch_types=[
            pltpu.VMEM((PIECE,), jnp.int32),
            pltpu.VMEM((D0,), jnp.int32),
            pltpu.VMEM((L,), jnp.int32),
            pltpu.VMEM((CPW,), jnp.int32),
            pltpu.VMEM((CPW,), jnp.int32),
        ],
    )
    def dedup(idx_hbm, jsrc_hbm, piece_v, table, shift_v, own_v, jbuf):
        cid = lax.axis_index("c")
        sid = lax.axis_index("s")
        wid = sid * NC + cid
        base = wid * CPW
        lanes = lax.iota(jnp.int32, L)
        last_lane = lanes == (L - 1)
        nxt_lane = jnp.minimum(lanes + 1, L - 1)

        def piece_body(p, _):
            pltpu.sync_copy(idx_hbm.at[pl.ds(p * PIECE, PIECE)], piece_v)

            def vreg_body(v, _):
                iv = piece_v[pl.ds(v * L, L)]
                pos = p * PIECE + v * L + lanes
                comb = (iv << POS_BITS) | pos
                ks, _ = plsc.sort_key_val(comb, comb)
                idx_s = ks >> POS_BITS
                pos_s = ks & POS_MASK
                shift_v[...] = idx_s
                nxt = plsc.load_gather(shift_v, [nxt_lane])
                keep = (idx_s != nxt) | last_lane
                plsc.store_scatter(table, [idx_s], pos_s, mask=keep)
                return 0

            lax.fori_loop(0, VPP, vreg_body, 0)
            return 0

        lax.fori_loop(0, NPIECE, piece_body, 0)

        pltpu.sync_copy(idx_hbm.at[pl.ds(base, CPW)], own_v)

        def src_body(c, _):
            iv = own_v[pl.ds(c * L, L)]
            jbuf[pl.ds(c * L, L)] = plsc.load_gather(table, [iv])
            return 0

        lax.fori_loop(0, CPW // L, src_body, 0)
        pltpu.sync_copy(jbuf, jsrc_hbm.at[pl.ds(base, CPW)])

    return dedup


def _make_transpose(K, R):
    info = plsc.get_sparse_core_info()
    NC, NS, L = info.num_cores, info.num_subcores, info.num_lanes
    NW = NC * NS
    CPW = K // NW
    TCH = 128
    NTCH = CPW // TCH
    RH = R // 2

    mesh = plsc.VectorSubcoreMesh(core_axis_name="c", subcore_axis_name="s")

    @functools.partial(
        pl.kernel,
        mesh=mesh,
        out_type=jax.ShapeDtypeStruct((K, R), jnp.float32),
        compiler_params=_SC_PARAMS,
        scratch_types=[
            pltpu.VMEM((RH, TCH), jnp.float32),
            pltpu.VMEM((TCH, R), jnp.float32),
        ],
    )
    def transpose(updt_hbm, upd_rm_hbm, tin, tout):
        cid = lax.axis_index("c")
        sid = lax.axis_index("s")
        wid = sid * NC + cid
        base = wid * CPW
        lanes = lax.iota(jnp.int32, L)

        def ch_body(ch, _):
            k0 = base + ch * TCH
            for rh in range(2):
                pltpu.sync_copy(
                    updt_hbm.at[pl.ds(rh * RH, RH), pl.ds(k0, TCH)], tin)

                def r_body(r, _):
                    rsplat = jnp.full((L,), rh * RH, jnp.int32) + r

                    def q_body(q, _):
                        vals = tin[r, pl.ds(q * L, L)]
                        plsc.store_scatter(tout, [q * L + lanes, rsplat], vals)
                        return 0

                    lax.fori_loop(0, TCH // L, q_body, 0, unroll=True)
                    return 0

                lax.fori_loop(0, RH, r_body, 0, unroll=4)
            pltpu.sync_copy(tout, upd_rm_hbm.at[pl.ds(k0, TCH)])
            return 0

        lax.fori_loop(0, NTCH, ch_body, 0)

    return transpose


def _make_scatter(D0, K, R):
    info = plsc.get_sparse_core_info()
    NC, NS, L = info.num_cores, info.num_subcores, info.num_lanes
    NW = NC * NS
    CPW = K // NW
    CHUNK = 64
    NCH = CPW // CHUNK

    mesh = plsc.VectorSubcoreMesh(core_axis_name="c", subcore_axis_name="s")

    @functools.partial(
        pl.kernel,
        mesh=mesh,
        out_type=(),
        compiler_params=_SC_PARAMS,
        scratch_types=[
            pltpu.VMEM((CPW,), jnp.int32),
            pltpu.VMEM((NCH, CHUNK), jnp.int32),
            pltpu.VMEM((NCH, CHUNK), jnp.int32),
            pltpu.VMEM((CHUNK, R), jnp.float32),
            pltpu.VMEM((CHUNK, R), jnp.float32),
            pltpu.SemaphoreType.DMA,
            pltpu.SemaphoreType.DMA,
            pltpu.SemaphoreType.DMA,
            pltpu.SemaphoreType.DMA,
        ],
    )
    def scatter(idx_hbm, jsrc_hbm, upd_hbm, out_hbm, stage_v, idx_own, jsrc,
                buf0, buf1, gsem0, gsem1, ssem0, ssem1):
        cid = lax.axis_index("c")
        sid = lax.axis_index("s")
        wid = sid * NC + cid
        base = wid * CPW

        pltpu.sync_copy(idx_hbm.at[pl.ds(base, CPW)], stage_v)

        def repack_idx(c, _):
            def repack_row(q, _):
                idx_own[c, pl.ds(q * L, L)] = stage_v[pl.ds(c * CHUNK + q * L, L)]
                return 0

            lax.fori_loop(0, CHUNK // L, repack_row, 0, unroll=True)
            return 0

        lax.fori_loop(0, NCH, repack_idx, 0)
        pltpu.sync_copy(jsrc_hbm.at[pl.ds(base, CPW)], stage_v)

        def repack_j(c, _):
            def repack_row(q, _):
                jsrc[c, pl.ds(q * L, L)] = stage_v[pl.ds(c * CHUNK + q * L, L)]
                return 0

            lax.fori_loop(0, CHUNK // L, repack_row, 0, unroll=True)
            return 0

        lax.fori_loop(0, NCH, repack_j, 0)

        bufs = (buf0, buf1)
        gsems = (gsem0, gsem1)
        ssems = (ssem0, ssem1)

        def pipe_body(cc, _):
            for b in range(2):
                c = cc * 2 + b

                @pl.when(cc > 0)
                def _():
                    pltpu.make_async_copy(
                        bufs[b], out_hbm.at[idx_own.at[0]], ssems[b]).wait()

                pltpu.async_copy(upd_hbm.at[jsrc.at[c]], bufs[b], gsems[b]).wait()
                pltpu.async_copy(bufs[b], out_hbm.at[idx_own.at[c]], ssems[b])
            return 0

        lax.fori_loop(0, NCH // 2, pipe_body, 0)
        for b in range(2):
            pltpu.make_async_copy(
                bufs[b], out_hbm.at[idx_own.at[0]], ssems[b]).wait()

    return scatter


def kernel(input, indices, update):
    D0, D1, D2 = input.shape
    K = indices.shape[0]
    R = D1 * D2
    jsrc = _make_dedup(D0, K)(indices)
    out_ref = jax.new_ref(input.reshape(D0, R))
    _make_scatter(D0, K, R)(indices, jsrc, update.reshape(K, R), out_ref)
    return jax.freeze(out_ref).reshape(D0, D1, D2)
